# Initial kernel scaffold; baseline (speedup 1.0000x reference)
import jax
import jax.numpy as jnp
from jax.experimental import pallas as pl


def kernel(train, batch_size, x, edge_index, y, W1, b1, W2, b2, Wg, bg, Wrb_w, Wrb_b, Wrr_w, Wrr_b, Wbb_w, Wbb_b, q_w, cls_w, cls_b):
    raise NotImplementedError("write your pallas kernel here")


# speedup vs baseline: 29.3194x; 29.3194x over previous
import functools

import jax
import jax.numpy as jnp
from jax import lax
from jax.experimental import pallas as pl
from jax.experimental.pallas import tpu as pltpu
from jax.experimental.pallas import tpu_sc as plsc

N = 10000
E = 320000
IN_CH = 128
HID = 96
DIM = HID // 3
BS = 4096

NC = 2
NS = 16
NW = NC * NS
CH = 128
EPW = 10240
NCHUNK = EPW // CH
ACC_ROWS = N + 240
TRASH = N
HP = 128

def _sc_hist_body(dsts_hbm, ones_hbm, zeros_hbm, out_hbm, dst_v, ones_v, obuf, hist_sh):
    cid = lax.axis_index("c")
    sid = lax.axis_index("s")
    wid = sid * NC + cid
    rows = ACC_ROWS // NS
    pltpu.sync_copy(dsts_hbm.at[wid], dst_v)
    pltpu.sync_copy(ones_hbm, ones_v)
    pltpu.sync_copy(zeros_hbm, obuf)
    pltpu.sync_copy(obuf, hist_sh.at[pl.ds(sid * rows, rows)])
    plsc.subcore_barrier()

    def body(j, _):
        pltpu.sync_copy(ones_v, hist_sh.at[dst_v.at[j]], add=True)
        return ()

    lax.fori_loop(0, NCHUNK, body, ())
    plsc.subcore_barrier()
    pltpu.sync_copy(hist_sh.at[pl.ds(sid * rows, rows)], obuf)
    pltpu.sync_copy(obuf, out_hbm.at[cid, pl.ds(sid * rows, rows)])


def _sc_scatter_body(xs_hbm, srcs_hbm, dsts_hbm, zeros_hbm, out_hbm,
                     src_v, dst_v, rbuf, acc_sh, sem):
    cid = lax.axis_index("c")
    sid = lax.axis_index("s")
    wid = sid * NC + cid
    pltpu.sync_copy(srcs_hbm.at[wid], src_v)
    pltpu.sync_copy(dsts_hbm.at[wid], dst_v)
    pltpu.sync_copy(zeros_hbm, rbuf)
    zrows = ACC_ROWS // NS
    for k in range(zrows // CH):
        pltpu.sync_copy(rbuf, acc_sh.at[pl.ds(sid * zrows + k * CH, CH)])
    plsc.subcore_barrier()

    def body(j, _):
        pltpu.async_copy(xs_hbm.at[src_v.at[j]], rbuf, sem).wait()
        pltpu.sync_copy(rbuf, acc_sh.at[dst_v.at[j]], add=True)
        return ()

    lax.fori_loop(0, NCHUNK, body, ())
    plsc.subcore_barrier()
    orows = BS // NS
    for k in range(orows // CH):
        pltpu.sync_copy(acc_sh.at[pl.ds(sid * orows + k * CH, CH)], rbuf)
        pltpu.sync_copy(rbuf, out_hbm.at[cid, pl.ds(sid * orows + k * CH, CH)])


@functools.cache
def _sc_kernels():
    mesh = plsc.VectorSubcoreMesh(core_axis_name="c", subcore_axis_name="s",
                                  num_cores=NC, num_subcores=NS)
    sc_hist = pl.kernel(
        _sc_hist_body,
        out_type=jax.ShapeDtypeStruct((NC, ACC_ROWS), jnp.float32),
        mesh=mesh,
        scratch_types=[
            pltpu.VMEM((NCHUNK, CH), jnp.int32),
            pltpu.VMEM((CH,), jnp.float32),
            pltpu.VMEM((ACC_ROWS // NS,), jnp.float32),
            pltpu.VMEM_SHARED((ACC_ROWS,), jnp.float32),
        ],
    )
    sc_scatter = pl.kernel(
        _sc_scatter_body,
        out_type=jax.ShapeDtypeStruct((NC, BS, HP), jnp.float32),
        mesh=mesh,
        scratch_types=[
            pltpu.VMEM((NCHUNK, CH), jnp.int32),
            pltpu.VMEM((NCHUNK, CH), jnp.int32),
            pltpu.VMEM((CH, HP), jnp.float32),
            pltpu.VMEM_SHARED((ACC_ROWS, HP), jnp.float32),
            pltpu.SemaphoreType.DMA,
        ],
    )
    return sc_hist, sc_scatter


def _tc1_body(x_ref, w1_ref, b1_ref, w2_ref, b2_ref, wg_ref, dp0_ref, dp1_ref,
              xs_ref, dinv_ref):
    x = x_ref[...]
    z = jnp.dot(x, w1_ref[...], preferred_element_type=jnp.float32) + b1_ref[...]
    h1 = jnp.where(z >= 0, z, 0.01 * z)
    h = jnp.dot(h1, w2_ref[...], preferred_element_type=jnp.float32) + b2_ref[...]
    xw = jnp.dot(h, wg_ref[...], preferred_element_type=jnp.float32)
    deg = dp0_ref[...] + dp1_ref[...] + 1.0
    dinv = lax.rsqrt(deg)
    xsp = jnp.concatenate(
        [xw * dinv, jnp.zeros((xw.shape[0], HP - HID), jnp.float32)], axis=1)
    xs_ref[...] = xsp
    dinv_ref[...] = dinv


def _tc1(x, W1, b1, W2, b2, Wg, dp0, dp1):
    R = 1000
    grid = (N // R,)
    return pl.pallas_call(
        _tc1_body,
        grid=grid,
        in_specs=[
            pl.BlockSpec((R, IN_CH), lambda i: (i, 0)),
            pl.BlockSpec((IN_CH, HID), lambda i: (0, 0)),
            pl.BlockSpec((1, HID), lambda i: (0, 0)),
            pl.BlockSpec((HID, HID), lambda i: (0, 0)),
            pl.BlockSpec((1, HID), lambda i: (0, 0)),
            pl.BlockSpec((HID, HID), lambda i: (0, 0)),
            pl.BlockSpec((R, 1), lambda i: (i, 0)),
            pl.BlockSpec((R, 1), lambda i: (i, 0)),
        ],
        out_specs=[
            pl.BlockSpec((R, HP), lambda i: (i, 0)),
            pl.BlockSpec((R, 1), lambda i: (i, 0)),
        ],
        out_shape=[
            jax.ShapeDtypeStruct((N, HP), jnp.float32),
            jax.ShapeDtypeStruct((N, 1), jnp.float32),
        ],
    )(x, W1, b1.reshape(1, HID), W2, b2.reshape(1, HID), Wg, dp0, dp1)


def _tc2_body(a0_ref, a1_ref, xs_ref, dinv_ref, bg_ref,
              wrb_ref, brb_ref, wrr_ref, brr_ref, wbb_ref, bbb_ref,
              qw_ref, cw_ref, cb_ref, yhat_ref, hb_ref):
    hb = (dinv_ref[...]
          * (a0_ref[...][:, :HID] + a1_ref[...][:, :HID] + xs_ref[...][:, :HID])
          + bg_ref[...])
    br = hb[:, :DIM]
    rr = hb[:, DIM:2 * DIM]
    bb = hb[:, 2 * DIM:]
    w0 = jnp.dot(jnp.tanh(br) @ wrb_ref[...] + brb_ref[...], qw_ref[...],
                 preferred_element_type=jnp.float32)
    w1 = jnp.dot(jnp.tanh(rr) @ wrr_ref[...] + brr_ref[...], qw_ref[...],
                 preferred_element_type=jnp.float32)
    w2 = jnp.dot(jnp.tanh(bb) @ wbb_ref[...] + bbb_ref[...], qw_ref[...],
                 preferred_element_type=jnp.float32)
    m = jnp.maximum(jnp.maximum(w0, w1), w2)
    e0 = jnp.exp(w0 - m)
    e1 = jnp.exp(w1 - m)
    e2 = jnp.exp(w2 - m)
    s = e0 + e1 + e2
    cx = (e0 * br + e1 * rr + e2 * bb) / s
    yhat_ref[...] = jnp.dot(cx, cw_ref[...], preferred_element_type=jnp.float32) + cb_ref[...]
    hb_ref[...] = hb


def _tc2(a0, a1, xsb, dinvb, bg, Wrb_w, Wrb_b, Wrr_w, Wrr_b, Wbb_w, Wbb_b,
         q_w, cls_w, cls_b):
    R = 1024
    grid = (BS // R,)
    full = lambda i: (0, 0)
    row = lambda i: (i, 0)
    return pl.pallas_call(
        _tc2_body,
        grid=grid,
        in_specs=[
            pl.BlockSpec((R, HP), row),
            pl.BlockSpec((R, HP), row),
            pl.BlockSpec((R, HP), row),
            pl.BlockSpec((R, 1), row),
            pl.BlockSpec((1, HID), full),
            pl.BlockSpec((DIM, DIM), full),
            pl.BlockSpec((1, DIM), full),
            pl.BlockSpec((DIM, DIM), full),
            pl.BlockSpec((1, DIM), full),
            pl.BlockSpec((DIM, DIM), full),
            pl.BlockSpec((1, DIM), full),
            pl.BlockSpec((DIM, 1), full),
            pl.BlockSpec((DIM, 2), full),
            pl.BlockSpec((1, 2), full),
        ],
        out_specs=[
            pl.BlockSpec((R, 2), row),
            pl.BlockSpec((R, HID), row),
        ],
        out_shape=[
            jax.ShapeDtypeStruct((BS, 2), jnp.float32),
            jax.ShapeDtypeStruct((BS, HID), jnp.float32),
        ],
    )(a0, a1, xsb, dinvb, bg.reshape(1, HID),
      Wrb_w, Wrb_b.reshape(1, DIM), Wrr_w, Wrr_b.reshape(1, DIM),
      Wbb_w, Wbb_b.reshape(1, DIM), q_w, cls_w, cls_b.reshape(1, 2))


def kernel(train, batch_size, x, edge_index, y, W1, b1, W2, b2, Wg, bg,
           Wrb_w, Wrb_b, Wrr_w, Wrr_b, Wbb_w, Wbb_b, q_w, cls_w, cls_b):
    pad_n = NW * EPW - E
    pad_src = (jnp.arange(pad_n, dtype=jnp.int32) * 131) % N
    src_p = jnp.concatenate([edge_index[0], pad_src]).reshape(NW, NCHUNK, CH)
    dst_p = jnp.concatenate(
        [edge_index[1], jnp.full((pad_n,), TRASH, jnp.int32)]
    ).reshape(NW, NCHUNK, CH)
    ones1 = jnp.ones((CH,), jnp.float32)
    zeros1 = jnp.zeros((ACC_ROWS // NS,), jnp.float32)
    zeros96 = jnp.zeros((CH, HP), jnp.float32)

    sc_hist, sc_scatter = _sc_kernels()
    hist = sc_hist(dst_p, ones1, zeros1)
    dp0 = hist[0, :N, None]
    dp1 = hist[1, :N, None]
    xs, dinv = _tc1(x, W1, b1, W2, b2, Wg, dp0, dp1)
    acc = sc_scatter(xs, src_p, dst_p, zeros96)
    yhat, hb = _tc2(acc[0], acc[1], xs[:BS], dinv[:BS], bg,
                    Wrb_w, Wrb_b, Wrr_w, Wrr_b, Wbb_w, Wbb_b,
                    q_w, cls_w, cls_b)
    return (yhat, hb[:, :DIM], hb[:, DIM:])


# speedup vs baseline: 39.2322x; 1.3381x over previous
import functools

import jax
import jax.numpy as jnp
from jax import lax
from jax.experimental import pallas as pl
from jax.experimental.pallas import tpu as pltpu
from jax.experimental.pallas import tpu_sc as plsc

N = 10000
E = 320000
IN_CH = 128
HID = 96
DIM = HID // 3
BS = 4096

NC = 2
NS = 16
NW = NC * NS
CH = 128
EPW = 10240
NCHUNK = EPW // CH
HIST_ROWS = N + 240
ACC_ROWS = BS + 128
TRASH = N
HP = 128

def _sc_hist_body(dsts_hbm, ones_hbm, zeros_hbm, out_hbm, dst_v, ones_v, obuf, hist_sh):
    cid = lax.axis_index("c")
    sid = lax.axis_index("s")
    wid = sid * NC + cid
    rows = HIST_ROWS // NS
    pltpu.sync_copy(dsts_hbm.at[wid], dst_v)
    pltpu.sync_copy(ones_hbm, ones_v)
    pltpu.sync_copy(zeros_hbm, obuf)
    pltpu.sync_copy(obuf, hist_sh.at[pl.ds(sid * rows, rows)])
    plsc.subcore_barrier()

    def body(j, _):
        pltpu.sync_copy(ones_v, hist_sh.at[dst_v.at[j]], add=True)
        return ()

    lax.fori_loop(0, NCHUNK, body, ())
    plsc.subcore_barrier()
    pltpu.sync_copy(hist_sh.at[pl.ds(sid * rows, rows)], obuf)
    pltpu.sync_copy(obuf, out_hbm.at[cid, pl.ds(sid * rows, rows)])


def _sc_scatter_body(xs_hbm, srcs_hbm, dsts_hbm, zeros_hbm, out_hbm,
                     src_v, dst_v, buf0, buf1, acc_sh, sem0, sem1):
    cid = lax.axis_index("c")
    sid = lax.axis_index("s")
    wid = sid * NC + cid
    pltpu.sync_copy(srcs_hbm.at[wid], src_v.at[pl.ds(0, NCHUNK)])
    pltpu.sync_copy(dsts_hbm.at[wid], dst_v)
    pltpu.sync_copy(zeros_hbm, buf0)
    zrows = ACC_ROWS // NS
    base = sid * zrows
    pltpu.sync_copy(buf0, acc_sh.at[pl.ds(base, CH)])
    pltpu.sync_copy(buf0, acc_sh.at[pl.ds(base + CH, CH)])
    pltpu.sync_copy(buf0.at[pl.ds(0, zrows - 2 * CH)],
                    acc_sh.at[pl.ds(base + 2 * CH, zrows - 2 * CH)])
    plsc.subcore_barrier()

    pltpu.sync_copy(srcs_hbm.at[wid, 0], src_v.at[NCHUNK])
    pltpu.sync_copy(srcs_hbm.at[wid, 0], src_v.at[NCHUNK + 1])

    pltpu.async_copy(xs_hbm.at[src_v.at[0]], buf0, sem0)
    pltpu.async_copy(xs_hbm.at[src_v.at[1]], buf1, sem1)

    def body(j, _):
        pltpu.make_async_copy(xs_hbm.at[src_v.at[j]], buf0, sem0).wait()
        pltpu.sync_copy(buf0, acc_sh.at[dst_v.at[j]], add=True)
        pltpu.async_copy(xs_hbm.at[src_v.at[j + 2]], buf0, sem0)
        pltpu.make_async_copy(xs_hbm.at[src_v.at[j + 1]], buf1, sem1).wait()
        pltpu.sync_copy(buf1, acc_sh.at[dst_v.at[j + 1]], add=True)
        pltpu.async_copy(xs_hbm.at[src_v.at[j + 3]], buf1, sem1)
        return ()

    lax.fori_loop(0, NCHUNK // 2, lambda i, c: body(2 * i, c), ())
    pltpu.make_async_copy(xs_hbm.at[src_v.at[0]], buf0, sem0).wait()
    pltpu.make_async_copy(xs_hbm.at[src_v.at[0]], buf1, sem1).wait()
    plsc.subcore_barrier()
    orows = BS // NS
    for k in range(orows // CH):
        pltpu.sync_copy(acc_sh.at[pl.ds(sid * orows + k * CH, CH)], buf0)
        pltpu.sync_copy(buf0, out_hbm.at[cid, pl.ds(sid * orows + k * CH, CH)])


@functools.cache
def _sc_kernels():
    mesh = plsc.VectorSubcoreMesh(core_axis_name="c", subcore_axis_name="s",
                                  num_cores=NC, num_subcores=NS)
    sc_hist = pl.kernel(
        _sc_hist_body,
        out_type=jax.ShapeDtypeStruct((NC, HIST_ROWS), jnp.float32),
        mesh=mesh,
        scratch_types=[
            pltpu.VMEM((NCHUNK, CH), jnp.int32),
            pltpu.VMEM((CH,), jnp.float32),
            pltpu.VMEM((HIST_ROWS // NS,), jnp.float32),
            pltpu.VMEM_SHARED((HIST_ROWS,), jnp.float32),
        ],
    )
    sc_scatter = pl.kernel(
        _sc_scatter_body,
        out_type=jax.ShapeDtypeStruct((NC, BS, HP), jnp.float32),
        mesh=mesh,
        scratch_types=[
            pltpu.VMEM((NCHUNK + 2, CH), jnp.int32),
            pltpu.VMEM((NCHUNK, CH), jnp.int32),
            pltpu.VMEM((CH, HP), jnp.float32),
            pltpu.VMEM((CH, HP), jnp.float32),
            pltpu.VMEM_SHARED((ACC_ROWS, HP), jnp.float32),
            pltpu.SemaphoreType.DMA,
            pltpu.SemaphoreType.DMA,
        ],
    )
    return sc_hist, sc_scatter


def _tc1_body(x_ref, w1_ref, b1_ref, w2_ref, b2_ref, wg_ref, dp0_ref, dp1_ref,
              xs_ref, dinv_ref):
    x = x_ref[...]
    z = jnp.dot(x, w1_ref[...], preferred_element_type=jnp.float32) + b1_ref[...]
    h1 = jnp.where(z >= 0, z, 0.01 * z)
    h = jnp.dot(h1, w2_ref[...], preferred_element_type=jnp.float32) + b2_ref[...]
    xw = jnp.dot(h, wg_ref[...], preferred_element_type=jnp.float32)
    deg = dp0_ref[...] + dp1_ref[...] + 1.0
    dinv = lax.rsqrt(deg)
    xsp = jnp.concatenate(
        [xw * dinv, jnp.zeros((xw.shape[0], HP - HID), jnp.float32)], axis=1)
    xs_ref[...] = xsp
    dinv_ref[...] = dinv


def _tc1(x, W1, b1, W2, b2, Wg, dp0, dp1):
    R = 1000
    grid = (N // R,)
    return pl.pallas_call(
        _tc1_body,
        grid=grid,
        in_specs=[
            pl.BlockSpec((R, IN_CH), lambda i: (i, 0)),
            pl.BlockSpec((IN_CH, HID), lambda i: (0, 0)),
            pl.BlockSpec((1, HID), lambda i: (0, 0)),
            pl.BlockSpec((HID, HID), lambda i: (0, 0)),
            pl.BlockSpec((1, HID), lambda i: (0, 0)),
            pl.BlockSpec((HID, HID), lambda i: (0, 0)),
            pl.BlockSpec((R, 1), lambda i: (i, 0)),
            pl.BlockSpec((R, 1), lambda i: (i, 0)),
        ],
        out_specs=[
            pl.BlockSpec((R, HP), lambda i: (i, 0)),
            pl.BlockSpec((R, 1), lambda i: (i, 0)),
        ],
        out_shape=[
            jax.ShapeDtypeStruct((N, HP), jnp.float32),
            jax.ShapeDtypeStruct((N, 1), jnp.float32),
        ],
    )(x, W1, b1.reshape(1, HID), W2, b2.reshape(1, HID), Wg, dp0, dp1)


def _tc2_body(a0_ref, a1_ref, xs_ref, dinv_ref, bg_ref,
              wrb_ref, brb_ref, wrr_ref, brr_ref, wbb_ref, bbb_ref,
              qw_ref, cw_ref, cb_ref, yhat_ref, hb_ref):
    hb = (dinv_ref[...]
          * (a0_ref[...][:, :HID] + a1_ref[...][:, :HID] + xs_ref[...][:, :HID])
          + bg_ref[...])
    br = hb[:, :DIM]
    rr = hb[:, DIM:2 * DIM]
    bb = hb[:, 2 * DIM:]
    w0 = jnp.dot(jnp.tanh(br) @ wrb_ref[...] + brb_ref[...], qw_ref[...],
                 preferred_element_type=jnp.float32)
    w1 = jnp.dot(jnp.tanh(rr) @ wrr_ref[...] + brr_ref[...], qw_ref[...],
                 preferred_element_type=jnp.float32)
    w2 = jnp.dot(jnp.tanh(bb) @ wbb_ref[...] + bbb_ref[...], qw_ref[...],
                 preferred_element_type=jnp.float32)
    m = jnp.maximum(jnp.maximum(w0, w1), w2)
    e0 = jnp.exp(w0 - m)
    e1 = jnp.exp(w1 - m)
    e2 = jnp.exp(w2 - m)
    s = e0 + e1 + e2
    cx = (e0 * br + e1 * rr + e2 * bb) / s
    yhat_ref[...] = jnp.dot(cx, cw_ref[...], preferred_element_type=jnp.float32) + cb_ref[...]
    hb_ref[...] = hb


def _tc2(a0, a1, xsb, dinvb, bg, Wrb_w, Wrb_b, Wrr_w, Wrr_b, Wbb_w, Wbb_b,
         q_w, cls_w, cls_b):
    R = 1024
    grid = (BS // R,)
    full = lambda i: (0, 0)
    row = lambda i: (i, 0)
    return pl.pallas_call(
        _tc2_body,
        grid=grid,
        in_specs=[
            pl.BlockSpec((R, HP), row),
            pl.BlockSpec((R, HP), row),
            pl.BlockSpec((R, HP), row),
            pl.BlockSpec((R, 1), row),
            pl.BlockSpec((1, HID), full),
            pl.BlockSpec((DIM, DIM), full),
            pl.BlockSpec((1, DIM), full),
            pl.BlockSpec((DIM, DIM), full),
            pl.BlockSpec((1, DIM), full),
            pl.BlockSpec((DIM, DIM), full),
            pl.BlockSpec((1, DIM), full),
            pl.BlockSpec((DIM, 1), full),
            pl.BlockSpec((DIM, 2), full),
            pl.BlockSpec((1, 2), full),
        ],
        out_specs=[
            pl.BlockSpec((R, 2), row),
            pl.BlockSpec((R, HID), row),
        ],
        out_shape=[
            jax.ShapeDtypeStruct((BS, 2), jnp.float32),
            jax.ShapeDtypeStruct((BS, HID), jnp.float32),
        ],
    )(a0, a1, xsb, dinvb, bg.reshape(1, HID),
      Wrb_w, Wrb_b.reshape(1, DIM), Wrr_w, Wrr_b.reshape(1, DIM),
      Wbb_w, Wbb_b.reshape(1, DIM), q_w, cls_w, cls_b.reshape(1, 2))


def kernel(train, batch_size, x, edge_index, y, W1, b1, W2, b2, Wg, bg,
           Wrb_w, Wrb_b, Wrr_w, Wrr_b, Wbb_w, Wbb_b, q_w, cls_w, cls_b):
    pad_n = NW * EPW - E
    pad_src = (jnp.arange(pad_n, dtype=jnp.int32) * 131) % N
    src_p = jnp.concatenate([edge_index[0], pad_src]).reshape(NW, NCHUNK, CH)
    dst_all = jnp.concatenate(
        [edge_index[1], jnp.full((pad_n,), TRASH, jnp.int32)])
    dst_p = dst_all.reshape(NW, NCHUNK, CH)
    spread = BS + (jnp.arange(NW * EPW, dtype=jnp.int32) % 128)
    dst_s = jnp.where(dst_all < BS, dst_all, spread).reshape(NW, NCHUNK, CH)
    ones1 = jnp.ones((CH,), jnp.float32)
    zeros1 = jnp.zeros((HIST_ROWS // NS,), jnp.float32)
    zeros96 = jnp.zeros((CH, HP), jnp.float32)

    sc_hist, sc_scatter = _sc_kernels()
    hist = sc_hist(dst_p, ones1, zeros1)
    dp0 = hist[0, :N, None]
    dp1 = hist[1, :N, None]
    xs, dinv = _tc1(x, W1, b1, W2, b2, Wg, dp0, dp1)
    acc = sc_scatter(xs, src_p, dst_s, zeros96)
    yhat, hb = _tc2(acc[0], acc[1], xs[:BS], dinv[:BS], bg,
                    Wrb_w, Wrb_b, Wrr_w, Wrr_b, Wbb_w, Wbb_b,
                    q_w, cls_w, cls_b)
    return (yhat, hb[:, :DIM], hb[:, DIM:])


# speedup vs baseline: 42.7001x; 1.0884x over previous
import functools

import jax
import jax.numpy as jnp
from jax import lax
from jax.experimental import pallas as pl
from jax.experimental.pallas import tpu as pltpu
from jax.experimental.pallas import tpu_sc as plsc

N = 10000
E = 320000
IN_CH = 128
HID = 96
DIM = HID // 3
BS = 4096

NC = 2
NS = 16
NW = NC * NS
CH = 128
EPW = 10240
NCHUNK = EPW // CH
HIST_ROWS = N + 240
ACC_ROWS = BS + 128
TRASH = N
HP = 128
DUMP = EPW + 768

def _sc_hist_body(dsts_hbm, ones_hbm, zeros_hbm, out_hbm, dst_v, ones_v, obuf, hist_sh):
    cid = lax.axis_index("c")
    sid = lax.axis_index("s")
    wid = sid * NC + cid
    rows = HIST_ROWS // NS
    pltpu.sync_copy(dsts_hbm.at[wid], dst_v)
    pltpu.sync_copy(ones_hbm, ones_v)
    pltpu.sync_copy(zeros_hbm, obuf)
    pltpu.sync_copy(obuf, hist_sh.at[pl.ds(sid * rows, rows)])
    plsc.subcore_barrier()

    def body(j, _):
        pltpu.sync_copy(ones_v, hist_sh.at[dst_v.at[j]], add=True)
        return ()

    lax.fori_loop(0, NCHUNK, body, ())
    plsc.subcore_barrier()
    pltpu.sync_copy(hist_sh.at[pl.ds(sid * rows, rows)], obuf)
    pltpu.sync_copy(obuf, out_hbm.at[cid, pl.ds(sid * rows, rows)])


FLEN = EPW + 512
FSZ = FLEN + 128
CHS = 64


def _sc_scatter_body(xs_hbm, srcs_hbm, dsts_hbm, zeros_hbm, zi_hbm, out_hbm,
                     src_v, dst_v, posb, fpkv, fsrc_v, fdst_v,
                     zbuf, buf0, buf1, sbuf, cnt_s, fpk_sh, acc_sh, sem0, sem1):
    cid = lax.axis_index("c")
    sid = lax.axis_index("s")
    wid = sid * NC + cid
    pltpu.sync_copy(srcs_hbm.at[wid], src_v)
    pltpu.sync_copy(dsts_hbm.at[wid], dst_v)
    pltpu.sync_copy(zeros_hbm, buf0)
    zrows = ACC_ROWS // NS
    zbase = sid * zrows
    for zk in range(4):
        pltpu.sync_copy(buf0, acc_sh.at[pl.ds(zbase + zk * CHS, CHS)])
    pltpu.sync_copy(buf0.at[pl.ds(0, zrows - 4 * CHS)],
                    acc_sh.at[pl.ds(zbase + 4 * CHS, zrows - 4 * CHS)])

    L = 16
    iota = lax.iota(jnp.int32, L)

    def p1(i, run):
        j = lax.shift_right_logical(i, 3)
        c = (i & (CH // L - 1)) * L
        d = dst_v[j, pl.ds(c, L)]
        m = lax.shift_right_logical(d - BS, 31)
        posb[pl.ds(i * L, L)] = run
        return run + m

    run = lax.fori_loop(0, EPW // L, p1, jnp.zeros((L,), jnp.int32))

    sbuf[pl.ds(0, L)] = iota * 0
    ps = run
    for sh in (1, 2, 4, 8):
        sbuf[pl.ds(L, L)] = ps
        ps = ps + sbuf[pl.ds(L - sh, L)]
    base_vec = ps - run + sid * FSZ
    sbuf[pl.ds(L, L)] = ps
    off = sbuf[pl.ds(L, L)][L - 1]

    def p2(i, _):
        j = lax.shift_right_logical(i, 3)
        c = (i & (CH // L - 1)) * L
        d = dst_v[j, pl.ds(c, L)]
        sv = src_v[j, pl.ds(c, L)]
        m = lax.shift_right_logical(d - BS, 31)
        pref = posb[pl.ds(i * L, L)]
        dump = sid * FSZ + FLEN + (i & (CH // L - 1)) * L + iota
        posb[pl.ds(i * L, L)] = m * (base_vec + pref) + (1 - m) * dump
        fpkv[pl.ds(i * L, L)] = d * 16384 + sv
        return ()

    lax.fori_loop(0, EPW // L, p2, ())

    pltpu.sync_copy(posb, fpk_sh.at[pl.ds(sid * FSZ, EPW)])
    pltpu.sync_copy(fpk_sh.at[pl.ds(sid * FSZ, EPW)], posb)
    pltpu.sync_copy(fpkv.at[pl.ds(0, EPW)], fpk_sh.at[pl.ds(sid * FSZ, EPW)])
    pltpu.sync_copy(fpk_sh.at[pl.ds(sid * FSZ, EPW)], fpkv.at[pl.ds(0, EPW)])

    pltpu.sync_copy(zi_hbm, zbuf)

    def pz(k, _):
        pltpu.sync_copy(zbuf, fpk_sh.at[pl.ds(sid * FSZ + k * CH, CH)])
        return ()

    lax.fori_loop(0, FSZ // CH, pz, ())

    def p3(j, _):
        q = j * CH
        pltpu.sync_copy(fpkv.at[pl.ds(q, CH)],
                        fpk_sh.at[posb.at[pl.ds(q, CH)]], add=True)
        return ()

    lax.fori_loop(0, NCHUNK, p3, ())
    plsc.subcore_barrier()
    pltpu.sync_copy(fpk_sh.at[pl.ds(sid * FSZ, FLEN)], fpkv)

    def p4(i, _):
        v = fpkv[pl.ds(i * L, L)]
        fsrc_v[pl.ds(i * L, L)] = jnp.minimum(v & 16383, N - 1)
        fdst_v[pl.ds(i * L, L)] = jnp.minimum(
            lax.shift_right_logical(v, 14), ACC_ROWS - 1)
        return ()

    lax.fori_loop(0, FLEN // L, p4, ())

    for k in range(512 // L):
        fsrc_v[pl.ds(off + k * L, L)] = (iota * 613 + k * 131) % N
        fdst_v[pl.ds(off + k * L, L)] = BS + ((iota + k * L) % CH)

    pltpu.sync_copy(fsrc_v, fpk_sh.at[pl.ds(sid * FSZ, FLEN)])
    pltpu.sync_copy(fpk_sh.at[pl.ds(sid * FSZ, FLEN)], fsrc_v)
    pltpu.sync_copy(fdst_v, fpk_sh.at[pl.ds(sid * FSZ, FLEN)])
    pltpu.sync_copy(fpk_sh.at[pl.ds(sid * FSZ, FLEN)], fdst_v)

    npairs = lax.shift_right_logical(off + 127, 7)

    plsc.subcore_barrier()

    pltpu.async_copy(xs_hbm.at[fsrc_v.at[pl.ds(0, CHS)]], buf0, sem0)
    pltpu.async_copy(xs_hbm.at[fsrc_v.at[pl.ds(CHS, CHS)]], buf1, sem1)

    def body(i, _):
        q = i * (2 * CHS)
        pltpu.make_async_copy(xs_hbm.at[fsrc_v.at[pl.ds(q, CHS)]], buf0, sem0).wait()
        pltpu.sync_copy(buf0, acc_sh.at[fdst_v.at[pl.ds(q, CHS)]], add=True)
        pltpu.async_copy(xs_hbm.at[fsrc_v.at[pl.ds(q + 2 * CHS, CHS)]], buf0, sem0)
        pltpu.make_async_copy(xs_hbm.at[fsrc_v.at[pl.ds(q + CHS, CHS)]], buf1, sem1).wait()
        pltpu.sync_copy(buf1, acc_sh.at[fdst_v.at[pl.ds(q + CHS, CHS)]], add=True)
        pltpu.async_copy(xs_hbm.at[fsrc_v.at[pl.ds(q + 3 * CHS, CHS)]], buf1, sem1)
        return ()

    lax.fori_loop(0, npairs, body, ())
    pltpu.make_async_copy(xs_hbm.at[fsrc_v.at[pl.ds(0, CHS)]], buf0, sem0).wait()
    pltpu.make_async_copy(xs_hbm.at[fsrc_v.at[pl.ds(0, CHS)]], buf1, sem1).wait()
    plsc.subcore_barrier()
    orows = BS // NS
    for k in range(orows // CHS):
        pltpu.sync_copy(acc_sh.at[pl.ds(sid * orows + k * CHS, CHS)], buf0)
        pltpu.sync_copy(buf0, out_hbm.at[cid, pl.ds(sid * orows + k * CHS, CHS)])


@functools.cache
def _sc_kernels():
    mesh = plsc.VectorSubcoreMesh(core_axis_name="c", subcore_axis_name="s",
                                  num_cores=NC, num_subcores=NS)
    sc_hist = pl.kernel(
        _sc_hist_body,
        out_type=jax.ShapeDtypeStruct((NC, HIST_ROWS), jnp.float32),
        mesh=mesh,
        scratch_types=[
            pltpu.VMEM((NCHUNK, CH), jnp.int32),
            pltpu.VMEM((CH,), jnp.float32),
            pltpu.VMEM((HIST_ROWS // NS,), jnp.float32),
            pltpu.VMEM_SHARED((HIST_ROWS,), jnp.float32),
        ],
    )
    sc_scatter = pl.kernel(
        _sc_scatter_body,
        out_type=jax.ShapeDtypeStruct((NC, BS, HP), jnp.float32),
        mesh=mesh,
        scratch_types=[
            pltpu.VMEM((NCHUNK, CH), jnp.int32),
            pltpu.VMEM((NCHUNK, CH), jnp.int32),
            pltpu.VMEM((EPW,), jnp.int32),
            pltpu.VMEM((FLEN,), jnp.int32),
            pltpu.VMEM((FLEN,), jnp.int32),
            pltpu.VMEM((FLEN,), jnp.int32),
            pltpu.VMEM((CH,), jnp.int32),
            pltpu.VMEM((CHS, HP), jnp.float32),
            pltpu.VMEM((CHS, HP), jnp.float32),
            pltpu.VMEM((2 * 16,), jnp.int32),
            pltpu.SMEM((16,), jnp.int32),
            pltpu.VMEM_SHARED((NS * FSZ,), jnp.int32),
            pltpu.VMEM_SHARED((ACC_ROWS, HP), jnp.float32),
            pltpu.SemaphoreType.DMA,
            pltpu.SemaphoreType.DMA,
        ],
    )
    return sc_hist, sc_scatter


def _tc1_body(x_ref, w1_ref, b1_ref, w2_ref, b2_ref, wg_ref, dp0_ref, dp1_ref,
              xs_ref, dinv_ref):
    x = x_ref[...]
    z = jnp.dot(x, w1_ref[...], preferred_element_type=jnp.float32) + b1_ref[...]
    h1 = jnp.where(z >= 0, z, 0.01 * z)
    h = jnp.dot(h1, w2_ref[...], preferred_element_type=jnp.float32) + b2_ref[...]
    xw = jnp.dot(h, wg_ref[...], preferred_element_type=jnp.float32)
    deg = dp0_ref[...] + dp1_ref[...] + 1.0
    dinv = lax.rsqrt(deg)
    xsp = jnp.concatenate(
        [xw * dinv, jnp.zeros((xw.shape[0], HP - HID), jnp.float32)], axis=1)
    xs_ref[...] = xsp
    dinv_ref[...] = dinv


def _tc1(x, W1, b1, W2, b2, Wg, dp0, dp1):
    R = 1000
    grid = (N // R,)
    return pl.pallas_call(
        _tc1_body,
        grid=grid,
        in_specs=[
            pl.BlockSpec((R, IN_CH), lambda i: (i, 0)),
            pl.BlockSpec((IN_CH, HID), lambda i: (0, 0)),
            pl.BlockSpec((1, HID), lambda i: (0, 0)),
            pl.BlockSpec((HID, HID), lambda i: (0, 0)),
            pl.BlockSpec((1, HID), lambda i: (0, 0)),
            pl.BlockSpec((HID, HID), lambda i: (0, 0)),
            pl.BlockSpec((R, 1), lambda i: (i, 0)),
            pl.BlockSpec((R, 1), lambda i: (i, 0)),
        ],
        out_specs=[
            pl.BlockSpec((R, HP), lambda i: (i, 0)),
            pl.BlockSpec((R, 1), lambda i: (i, 0)),
        ],
        out_shape=[
            jax.ShapeDtypeStruct((N, HP), jnp.float32),
            jax.ShapeDtypeStruct((N, 1), jnp.float32),
        ],
    )(x, W1, b1.reshape(1, HID), W2, b2.reshape(1, HID), Wg, dp0, dp1)


def _tc2_body(a0_ref, a1_ref, xs_ref, dinv_ref, bg_ref,
              wrb_ref, brb_ref, wrr_ref, brr_ref, wbb_ref, bbb_ref,
              qw_ref, cw_ref, cb_ref, yhat_ref, hb_ref):
    hb = (dinv_ref[...]
          * (a0_ref[...][:, :HID] + a1_ref[...][:, :HID] + xs_ref[...][:, :HID])
          + bg_ref[...])
    br = hb[:, :DIM]
    rr = hb[:, DIM:2 * DIM]
    bb = hb[:, 2 * DIM:]
    w0 = jnp.dot(jnp.tanh(br) @ wrb_ref[...] + brb_ref[...], qw_ref[...],
                 preferred_element_type=jnp.float32)
    w1 = jnp.dot(jnp.tanh(rr) @ wrr_ref[...] + brr_ref[...], qw_ref[...],
                 preferred_element_type=jnp.float32)
    w2 = jnp.dot(jnp.tanh(bb) @ wbb_ref[...] + bbb_ref[...], qw_ref[...],
                 preferred_element_type=jnp.float32)
    m = jnp.maximum(jnp.maximum(w0, w1), w2)
    e0 = jnp.exp(w0 - m)
    e1 = jnp.exp(w1 - m)
    e2 = jnp.exp(w2 - m)
    s = e0 + e1 + e2
    cx = (e0 * br + e1 * rr + e2 * bb) / s
    yhat_ref[...] = jnp.dot(cx, cw_ref[...], preferred_element_type=jnp.float32) + cb_ref[...]
    hb_ref[...] = hb


def _tc2(a0, a1, xsb, dinvb, bg, Wrb_w, Wrb_b, Wrr_w, Wrr_b, Wbb_w, Wbb_b,
         q_w, cls_w, cls_b):
    R = 1024
    grid = (BS // R,)
    full = lambda i: (0, 0)
    row = lambda i: (i, 0)
    return pl.pallas_call(
        _tc2_body,
        grid=grid,
        in_specs=[
            pl.BlockSpec((R, HP), row),
            pl.BlockSpec((R, HP), row),
            pl.BlockSpec((R, HP), row),
            pl.BlockSpec((R, 1), row),
            pl.BlockSpec((1, HID), full),
            pl.BlockSpec((DIM, DIM), full),
            pl.BlockSpec((1, DIM), full),
            pl.BlockSpec((DIM, DIM), full),
            pl.BlockSpec((1, DIM), full),
            pl.BlockSpec((DIM, DIM), full),
            pl.BlockSpec((1, DIM), full),
            pl.BlockSpec((DIM, 1), full),
            pl.BlockSpec((DIM, 2), full),
            pl.BlockSpec((1, 2), full),
        ],
        out_specs=[
            pl.BlockSpec((R, 2), row),
            pl.BlockSpec((R, HID), row),
        ],
        out_shape=[
            jax.ShapeDtypeStruct((BS, 2), jnp.float32),
            jax.ShapeDtypeStruct((BS, HID), jnp.float32),
        ],
    )(a0, a1, xsb, dinvb, bg.reshape(1, HID),
      Wrb_w, Wrb_b.reshape(1, DIM), Wrr_w, Wrr_b.reshape(1, DIM),
      Wbb_w, Wbb_b.reshape(1, DIM), q_w, cls_w, cls_b.reshape(1, 2))


def kernel(train, batch_size, x, edge_index, y, W1, b1, W2, b2, Wg, bg,
           Wrb_w, Wrb_b, Wrr_w, Wrr_b, Wbb_w, Wbb_b, q_w, cls_w, cls_b):
    pad_n = NW * EPW - E
    pad_src = (jnp.arange(pad_n, dtype=jnp.int32) * 131) % N
    src_p = jnp.concatenate([edge_index[0], pad_src]).reshape(NW, NCHUNK, CH)
    dst_all = jnp.concatenate(
        [edge_index[1], jnp.full((pad_n,), TRASH, jnp.int32)])
    dst_p = dst_all.reshape(NW, NCHUNK, CH)
    spread = BS + (jnp.arange(NW * EPW, dtype=jnp.int32) % 128)
    dst_s = jnp.where(dst_all < BS, dst_all, spread).reshape(NW, NCHUNK, CH)
    ones1 = jnp.ones((CH,), jnp.float32)
    zeros1 = jnp.zeros((HIST_ROWS // NS,), jnp.float32)
    zeros96 = jnp.zeros((CHS, HP), jnp.float32)
    zi32 = jnp.zeros((CH,), jnp.int32)

    sc_hist, sc_scatter = _sc_kernels()
    hist = sc_hist(dst_p, ones1, zeros1)
    dp0 = hist[0, :N, None]
    dp1 = hist[1, :N, None]
    xs, dinv = _tc1(x, W1, b1, W2, b2, Wg, dp0, dp1)
    acc = sc_scatter(xs, src_p, dst_s, zeros96, zi32)
    yhat, hb = _tc2(acc[0], acc[1], xs[:BS], dinv[:BS], bg,
                    Wrb_w, Wrb_b, Wrr_w, Wrr_b, Wbb_w, Wbb_b,
                    q_w, cls_w, cls_b)
    return (yhat, hb[:, :DIM], hb[:, DIM:])


# speedup vs baseline: 43.7509x; 1.0246x over previous
import functools

import jax
import jax.numpy as jnp
from jax import lax
from jax.experimental import pallas as pl
from jax.experimental.pallas import tpu as pltpu
from jax.experimental.pallas import tpu_sc as plsc

N = 10000
E = 320000
IN_CH = 128
HID = 96
DIM = HID // 3
BS = 4096

NC = 2
NS = 16
NW = NC * NS
CH = 128
EPW = 10240
NCHUNK = EPW // CH
HIST_ROWS = N + 240
ACC_ROWS = BS + 128
TRASH = N
HP = 128
DUMP = EPW + 768

def _sc_hist_body(dsts_hbm, ones_hbm, zeros_hbm, out_hbm, dst_v, ones_v, obuf, hist_sh):
    cid = lax.axis_index("c")
    sid = lax.axis_index("s")
    wid = sid * NC + cid
    rows = HIST_ROWS // NS
    pltpu.sync_copy(dsts_hbm.at[wid], dst_v)
    pltpu.sync_copy(ones_hbm, ones_v)
    pltpu.sync_copy(zeros_hbm, obuf)
    pltpu.sync_copy(obuf, hist_sh.at[pl.ds(sid * rows, rows)])
    plsc.subcore_barrier()

    def body(j, _):
        pltpu.sync_copy(ones_v, hist_sh.at[dst_v.at[j]], add=True)
        return ()

    lax.fori_loop(0, NCHUNK, body, ())
    plsc.subcore_barrier()
    pltpu.sync_copy(hist_sh.at[pl.ds(sid * rows, rows)], obuf)
    pltpu.sync_copy(obuf, out_hbm.at[cid, pl.ds(sid * rows, rows)])


FLEN = EPW + 512
FSZ = FLEN + 128
CHS = 64
ZB = 2176


def _sc_scatter_body(xs_hbm, srcs_hbm, dsts_hbm, zeros_hbm, zi_hbm, out_hbm,
                     src_v, dst_v, posb, fpkv, fsrc_v, fdst_v,
                     zbuf, buf0, buf1, sbuf, cnt_s, fpk_sh, acc_sh, sem0, sem1):
    cid = lax.axis_index("c")
    sid = lax.axis_index("s")
    wid = sid * NC + cid
    pltpu.sync_copy(srcs_hbm.at[wid], src_v)
    pltpu.sync_copy(dsts_hbm.at[wid], dst_v)
    pltpu.sync_copy(zeros_hbm, buf0)
    zrows = ACC_ROWS // NS
    zbase = sid * zrows
    for zk in range(4):
        pltpu.sync_copy(buf0, acc_sh.at[pl.ds(zbase + zk * CHS, CHS)])
    pltpu.sync_copy(buf0.at[pl.ds(0, zrows - 4 * CHS)],
                    acc_sh.at[pl.ds(zbase + 4 * CHS, zrows - 4 * CHS)])

    L = 16
    iota = lax.iota(jnp.int32, L)

    def p1(i, run):
        j = lax.shift_right_logical(i, 3)
        c = (i & (CH // L - 1)) * L
        d = dst_v[j, pl.ds(c, L)]
        m = lax.shift_right_logical(d - BS, 31)
        posb[pl.ds(i * L, L)] = run
        return run + m

    run = lax.fori_loop(0, EPW // L, p1, jnp.zeros((L,), jnp.int32))

    sbuf[pl.ds(0, L)] = iota * 0
    ps = run
    for sh in (1, 2, 4, 8):
        sbuf[pl.ds(L, L)] = ps
        ps = ps + sbuf[pl.ds(L - sh, L)]
    base_vec = ps - run + sid * FSZ
    sbuf[pl.ds(L, L)] = ps
    off = sbuf[pl.ds(L, L)][L - 1]

    def p2(i, _):
        j = lax.shift_right_logical(i, 3)
        c = (i & (CH // L - 1)) * L
        d = dst_v[j, pl.ds(c, L)]
        sv = src_v[j, pl.ds(c, L)]
        m = lax.shift_right_logical(d - BS, 31)
        pref = posb[pl.ds(i * L, L)]
        dump = sid * FSZ + FLEN + (i & (CH // L - 1)) * L + iota
        posb[pl.ds(i * L, L)] = m * (base_vec + pref) + (1 - m) * dump
        fpkv[pl.ds(i * L, L)] = d * 16384 + sv
        return ()

    lax.fori_loop(0, EPW // L, p2, ())

    pltpu.sync_copy(posb, fpk_sh.at[pl.ds(sid * FSZ, EPW)])
    pltpu.sync_copy(fpk_sh.at[pl.ds(sid * FSZ, EPW)], posb)
    pltpu.sync_copy(fpkv.at[pl.ds(0, EPW)], fpk_sh.at[pl.ds(sid * FSZ, EPW)])
    pltpu.sync_copy(fpk_sh.at[pl.ds(sid * FSZ, EPW)], fpkv.at[pl.ds(0, EPW)])

    pltpu.sync_copy(zi_hbm, zbuf)

    def pz(k, _):
        pltpu.sync_copy(zbuf, fpk_sh.at[pl.ds(sid * FSZ + k * ZB, ZB)])
        return ()

    lax.fori_loop(0, FSZ // ZB, pz, ())

    def p3(j, _):
        q = j * CH
        pltpu.sync_copy(fpkv.at[pl.ds(q, CH)],
                        fpk_sh.at[posb.at[pl.ds(q, CH)]], add=True)
        return ()

    lax.fori_loop(0, NCHUNK, p3, ())
    plsc.subcore_barrier()
    pltpu.sync_copy(fpk_sh.at[pl.ds(sid * FSZ, FLEN)], fpkv)

    def p4(i, _):
        v = fpkv[pl.ds(i * L, L)]
        fsrc_v[pl.ds(i * L, L)] = jnp.minimum(v & 16383, N - 1)
        fdst_v[pl.ds(i * L, L)] = jnp.minimum(
            lax.shift_right_logical(v, 14), ACC_ROWS - 1)
        return ()

    lax.fori_loop(0, FLEN // L, p4, ())

    for k in range(512 // L):
        fsrc_v[pl.ds(off + k * L, L)] = (iota * 613 + k * 131) % N
        fdst_v[pl.ds(off + k * L, L)] = BS + ((iota + k * L) % CH)

    pltpu.sync_copy(fsrc_v, fpk_sh.at[pl.ds(sid * FSZ, FLEN)])
    pltpu.sync_copy(fpk_sh.at[pl.ds(sid * FSZ, FLEN)], fsrc_v)
    pltpu.sync_copy(fdst_v, fpk_sh.at[pl.ds(sid * FSZ, FLEN)])
    pltpu.sync_copy(fpk_sh.at[pl.ds(sid * FSZ, FLEN)], fdst_v)

    npairs = lax.shift_right_logical(off + 127, 7)

    plsc.subcore_barrier()

    pltpu.async_copy(xs_hbm.at[fsrc_v.at[pl.ds(0, CHS)]], buf0, sem0)
    pltpu.async_copy(xs_hbm.at[fsrc_v.at[pl.ds(CHS, CHS)]], buf1, sem1)

    def body(i, _):
        q = i * (2 * CHS)
        pltpu.make_async_copy(xs_hbm.at[fsrc_v.at[pl.ds(q, CHS)]], buf0, sem0).wait()
        pltpu.sync_copy(buf0, acc_sh.at[fdst_v.at[pl.ds(q, CHS)]], add=True)
        pltpu.async_copy(xs_hbm.at[fsrc_v.at[pl.ds(q + 2 * CHS, CHS)]], buf0, sem0)
        pltpu.make_async_copy(xs_hbm.at[fsrc_v.at[pl.ds(q + CHS, CHS)]], buf1, sem1).wait()
        pltpu.sync_copy(buf1, acc_sh.at[fdst_v.at[pl.ds(q + CHS, CHS)]], add=True)
        pltpu.async_copy(xs_hbm.at[fsrc_v.at[pl.ds(q + 3 * CHS, CHS)]], buf1, sem1)
        return ()

    lax.fori_loop(0, npairs, body, ())
    pltpu.make_async_copy(xs_hbm.at[fsrc_v.at[pl.ds(0, CHS)]], buf0, sem0).wait()
    pltpu.make_async_copy(xs_hbm.at[fsrc_v.at[pl.ds(0, CHS)]], buf1, sem1).wait()
    plsc.subcore_barrier()
    orows = BS // NS
    for k in range(orows // CHS):
        pltpu.sync_copy(acc_sh.at[pl.ds(sid * orows + k * CHS, CHS)], buf0)
        pltpu.sync_copy(buf0, out_hbm.at[cid, pl.ds(sid * orows + k * CHS, CHS)])


@functools.cache
def _sc_kernels():
    mesh = plsc.VectorSubcoreMesh(core_axis_name="c", subcore_axis_name="s",
                                  num_cores=NC, num_subcores=NS)
    sc_hist = pl.kernel(
        _sc_hist_body,
        out_type=jax.ShapeDtypeStruct((NC, HIST_ROWS), jnp.float32),
        mesh=mesh,
        scratch_types=[
            pltpu.VMEM((NCHUNK, CH), jnp.int32),
            pltpu.VMEM((CH,), jnp.float32),
            pltpu.VMEM((HIST_ROWS // NS,), jnp.float32),
            pltpu.VMEM_SHARED((HIST_ROWS,), jnp.float32),
        ],
    )
    sc_scatter = pl.kernel(
        _sc_scatter_body,
        out_type=jax.ShapeDtypeStruct((NC, BS, HP), jnp.float32),
        mesh=mesh,
        scratch_types=[
            pltpu.VMEM((NCHUNK, CH), jnp.int32),
            pltpu.VMEM((NCHUNK, CH), jnp.int32),
            pltpu.VMEM((EPW,), jnp.int32),
            pltpu.VMEM((FLEN,), jnp.int32),
            pltpu.VMEM((FLEN,), jnp.int32),
            pltpu.VMEM((FLEN,), jnp.int32),
            pltpu.VMEM((ZB,), jnp.int32),
            pltpu.VMEM((CHS, HP), jnp.float32),
            pltpu.VMEM((CHS, HP), jnp.float32),
            pltpu.VMEM((2 * 16,), jnp.int32),
            pltpu.SMEM((16,), jnp.int32),
            pltpu.VMEM_SHARED((NS * FSZ,), jnp.int32),
            pltpu.VMEM_SHARED((ACC_ROWS, HP), jnp.float32),
            pltpu.SemaphoreType.DMA,
            pltpu.SemaphoreType.DMA,
        ],
    )
    return sc_hist, sc_scatter


def _tc1_body(x_ref, w1_ref, b1_ref, w2_ref, b2_ref, wg_ref, dp0_ref, dp1_ref,
              xs_ref, dinv_ref):
    x = x_ref[...]
    z = jnp.dot(x, w1_ref[...], preferred_element_type=jnp.float32) + b1_ref[...]
    h1 = jnp.where(z >= 0, z, 0.01 * z)
    h = jnp.dot(h1, w2_ref[...], preferred_element_type=jnp.float32) + b2_ref[...]
    xw = jnp.dot(h, wg_ref[...], preferred_element_type=jnp.float32)
    deg = dp0_ref[...] + dp1_ref[...] + 1.0
    dinv = lax.rsqrt(deg)
    xsp = jnp.concatenate(
        [xw * dinv, jnp.zeros((xw.shape[0], HP - HID), jnp.float32)], axis=1)
    xs_ref[...] = xsp
    dinv_ref[...] = dinv


def _tc1(x, W1, b1, W2, b2, Wg, dp0, dp1):
    R = 1000
    grid = (N // R,)
    return pl.pallas_call(
        _tc1_body,
        grid=grid,
        in_specs=[
            pl.BlockSpec((R, IN_CH), lambda i: (i, 0)),
            pl.BlockSpec((IN_CH, HID), lambda i: (0, 0)),
            pl.BlockSpec((1, HID), lambda i: (0, 0)),
            pl.BlockSpec((HID, HID), lambda i: (0, 0)),
            pl.BlockSpec((1, HID), lambda i: (0, 0)),
            pl.BlockSpec((HID, HID), lambda i: (0, 0)),
            pl.BlockSpec((R, 1), lambda i: (i, 0)),
            pl.BlockSpec((R, 1), lambda i: (i, 0)),
        ],
        out_specs=[
            pl.BlockSpec((R, HP), lambda i: (i, 0)),
            pl.BlockSpec((R, 1), lambda i: (i, 0)),
        ],
        out_shape=[
            jax.ShapeDtypeStruct((N, HP), jnp.float32),
            jax.ShapeDtypeStruct((N, 1), jnp.float32),
        ],
    )(x, W1, b1.reshape(1, HID), W2, b2.reshape(1, HID), Wg, dp0, dp1)


def _tc2_body(a0_ref, a1_ref, xs_ref, dinv_ref, bg_ref,
              wrb_ref, brb_ref, wrr_ref, brr_ref, wbb_ref, bbb_ref,
              qw_ref, cw_ref, cb_ref, yhat_ref, hb_ref):
    hb = (dinv_ref[...]
          * (a0_ref[...][:, :HID] + a1_ref[...][:, :HID] + xs_ref[...][:, :HID])
          + bg_ref[...])
    br = hb[:, :DIM]
    rr = hb[:, DIM:2 * DIM]
    bb = hb[:, 2 * DIM:]
    w0 = jnp.dot(jnp.tanh(br) @ wrb_ref[...] + brb_ref[...], qw_ref[...],
                 preferred_element_type=jnp.float32)
    w1 = jnp.dot(jnp.tanh(rr) @ wrr_ref[...] + brr_ref[...], qw_ref[...],
                 preferred_element_type=jnp.float32)
    w2 = jnp.dot(jnp.tanh(bb) @ wbb_ref[...] + bbb_ref[...], qw_ref[...],
                 preferred_element_type=jnp.float32)
    m = jnp.maximum(jnp.maximum(w0, w1), w2)
    e0 = jnp.exp(w0 - m)
    e1 = jnp.exp(w1 - m)
    e2 = jnp.exp(w2 - m)
    s = e0 + e1 + e2
    cx = (e0 * br + e1 * rr + e2 * bb) / s
    yhat_ref[...] = jnp.dot(cx, cw_ref[...], preferred_element_type=jnp.float32) + cb_ref[...]
    hb_ref[...] = hb


def _tc2(a0, a1, xsb, dinvb, bg, Wrb_w, Wrb_b, Wrr_w, Wrr_b, Wbb_w, Wbb_b,
         q_w, cls_w, cls_b):
    R = 1024
    grid = (BS // R,)
    full = lambda i: (0, 0)
    row = lambda i: (i, 0)
    return pl.pallas_call(
        _tc2_body,
        grid=grid,
        in_specs=[
            pl.BlockSpec((R, HP), row),
            pl.BlockSpec((R, HP), row),
            pl.BlockSpec((R, HP), row),
            pl.BlockSpec((R, 1), row),
            pl.BlockSpec((1, HID), full),
            pl.BlockSpec((DIM, DIM), full),
            pl.BlockSpec((1, DIM), full),
            pl.BlockSpec((DIM, DIM), full),
            pl.BlockSpec((1, DIM), full),
            pl.BlockSpec((DIM, DIM), full),
            pl.BlockSpec((1, DIM), full),
            pl.BlockSpec((DIM, 1), full),
            pl.BlockSpec((DIM, 2), full),
            pl.BlockSpec((1, 2), full),
        ],
        out_specs=[
            pl.BlockSpec((R, 2), row),
            pl.BlockSpec((R, HID), row),
        ],
        out_shape=[
            jax.ShapeDtypeStruct((BS, 2), jnp.float32),
            jax.ShapeDtypeStruct((BS, HID), jnp.float32),
        ],
    )(a0, a1, xsb, dinvb, bg.reshape(1, HID),
      Wrb_w, Wrb_b.reshape(1, DIM), Wrr_w, Wrr_b.reshape(1, DIM),
      Wbb_w, Wbb_b.reshape(1, DIM), q_w, cls_w, cls_b.reshape(1, 2))


def kernel(train, batch_size, x, edge_index, y, W1, b1, W2, b2, Wg, bg,
           Wrb_w, Wrb_b, Wrr_w, Wrr_b, Wbb_w, Wbb_b, q_w, cls_w, cls_b):
    pad_n = NW * EPW - E
    pad_src = (jnp.arange(pad_n, dtype=jnp.int32) * 131) % N
    src_p = jnp.concatenate([edge_index[0], pad_src]).reshape(NW, NCHUNK, CH)
    dst_all = jnp.concatenate(
        [edge_index[1], jnp.full((pad_n,), TRASH, jnp.int32)])
    dst_p = dst_all.reshape(NW, NCHUNK, CH)
    spread = BS + (jnp.arange(NW * EPW, dtype=jnp.int32) % 128)
    dst_s = jnp.where(dst_all < BS, dst_all, spread).reshape(NW, NCHUNK, CH)
    ones1 = jnp.ones((CH,), jnp.float32)
    zeros1 = jnp.zeros((HIST_ROWS // NS,), jnp.float32)
    zeros96 = jnp.zeros((CHS, HP), jnp.float32)
    zi32 = jnp.zeros((ZB,), jnp.int32)

    sc_hist, sc_scatter = _sc_kernels()
    hist = sc_hist(dst_p, ones1, zeros1)
    dp0 = hist[0, :N, None]
    dp1 = hist[1, :N, None]
    xs, dinv = _tc1(x, W1, b1, W2, b2, Wg, dp0, dp1)
    acc = sc_scatter(xs, src_p, dst_s, zeros96, zi32)
    yhat, hb = _tc2(acc[0], acc[1], xs[:BS], dinv[:BS], bg,
                    Wrb_w, Wrb_b, Wrr_w, Wrr_b, Wbb_w, Wbb_b,
                    q_w, cls_w, cls_b)
    return (yhat, hb[:, :DIM], hb[:, DIM:])


# speedup vs baseline: 44.4052x; 1.0150x over previous
import functools

import jax
import jax.numpy as jnp
from jax import lax
from jax.experimental import pallas as pl
from jax.experimental.pallas import tpu as pltpu
from jax.experimental.pallas import tpu_sc as plsc

N = 10000
E = 320000
IN_CH = 128
HID = 96
DIM = HID // 3
BS = 4096

NC = 2
NS = 16
NW = NC * NS
CH = 128
EPW = 10240
NCHUNK = EPW // CH
HIST_ROWS = N + 240
ACC_ROWS = BS + 128
TRASH = N
HP = 128
DUMP = EPW + 768

def _sc_hist_body(dsts_hbm, ones_hbm, zeros_hbm, out_hbm, dst_v, ones_v, obuf, hist_sh):
    cid = lax.axis_index("c")
    sid = lax.axis_index("s")
    wid = sid * NC + cid
    rows = HIST_ROWS // NS
    pltpu.sync_copy(dsts_hbm.at[wid], dst_v)
    pltpu.sync_copy(ones_hbm, ones_v)
    pltpu.sync_copy(zeros_hbm, obuf)
    pltpu.sync_copy(obuf, hist_sh.at[pl.ds(sid * rows, rows)])
    plsc.subcore_barrier()

    def body(j, _):
        pltpu.sync_copy(ones_v, hist_sh.at[dst_v.at[j]], add=True)
        return ()

    lax.fori_loop(0, NCHUNK, body, ())
    plsc.subcore_barrier()
    pltpu.sync_copy(hist_sh.at[pl.ds(sid * rows, rows)], obuf)
    pltpu.sync_copy(obuf, out_hbm.at[cid, pl.ds(sid * rows, rows)])


FLEN = EPW + 512
FSZ = FLEN + 128
CHS = 64
ZB = 2176


def _sc_scatter_body(xs_hbm, srcs_hbm, dsts_hbm, zeros_hbm, zi_hbm, out_hbm,
                     src_v, dst_v, posb, fpkv, fsrc_v, fdst_v,
                     zbuf, buf0, buf1, sbuf, cnt_s, fpk_sh, acc_sh, sem0, sem1):
    cid = lax.axis_index("c")
    sid = lax.axis_index("s")
    wid = sid * NC + cid
    pltpu.sync_copy(srcs_hbm.at[wid], src_v)
    pltpu.sync_copy(dsts_hbm.at[wid], dst_v)
    pltpu.sync_copy(zeros_hbm, buf0)
    zrows = ACC_ROWS // NS
    zbase = sid * zrows
    for zk in range(4):
        pltpu.sync_copy(buf0, acc_sh.at[pl.ds(zbase + zk * CHS, CHS)])
    pltpu.sync_copy(buf0.at[pl.ds(0, zrows - 4 * CHS)],
                    acc_sh.at[pl.ds(zbase + 4 * CHS, zrows - 4 * CHS)])

    L = 16
    iota = lax.iota(jnp.int32, L)

    def p1(i, run):
        j = lax.shift_right_logical(i, 3)
        c = (i & (CH // L - 1)) * L
        d = dst_v[j, pl.ds(c, L)]
        m = lax.shift_right_logical(d - BS, 31)
        posb[pl.ds(i * L, L)] = run
        return run + m

    run = lax.fori_loop(0, EPW // L, p1, jnp.zeros((L,), jnp.int32))

    sbuf[pl.ds(0, L)] = iota * 0
    ps = run
    for sh in (1, 2, 4, 8):
        sbuf[pl.ds(L, L)] = ps
        ps = ps + sbuf[pl.ds(L - sh, L)]
    base_vec = ps - run + sid * FSZ
    sbuf[pl.ds(L, L)] = ps
    off = sbuf[pl.ds(L, L)][L - 1]

    def p2(i, _):
        j = lax.shift_right_logical(i, 3)
        c = (i & (CH // L - 1)) * L
        d = dst_v[j, pl.ds(c, L)]
        sv = src_v[j, pl.ds(c, L)]
        m = lax.shift_right_logical(d - BS, 31)
        pref = posb[pl.ds(i * L, L)]
        dump = sid * FSZ + FLEN + (i & (CH // L - 1)) * L + iota
        posb[pl.ds(i * L, L)] = m * (base_vec + pref) + (1 - m) * dump
        fpkv[pl.ds(i * L, L)] = d * 16384 + sv
        return ()

    lax.fori_loop(0, EPW // L, p2, ())

    pltpu.sync_copy(posb, fpk_sh.at[pl.ds(sid * FSZ, EPW)])
    pltpu.sync_copy(fpk_sh.at[pl.ds(sid * FSZ, EPW)], posb)
    pltpu.sync_copy(fpkv.at[pl.ds(0, EPW)], fpk_sh.at[pl.ds(sid * FSZ, EPW)])
    pltpu.sync_copy(fpk_sh.at[pl.ds(sid * FSZ, EPW)], fpkv.at[pl.ds(0, EPW)])

    pltpu.sync_copy(zi_hbm, zbuf)

    def pz(k, _):
        pltpu.sync_copy(zbuf, fpk_sh.at[pl.ds(sid * FSZ + k * ZB, ZB)])
        return ()

    lax.fori_loop(0, FSZ // ZB, pz, ())

    def p3(j, _):
        q = j * CH
        pltpu.sync_copy(fpkv.at[pl.ds(q, CH)],
                        fpk_sh.at[posb.at[pl.ds(q, CH)]], add=True)
        return ()

    lax.fori_loop(0, NCHUNK, p3, ())
    plsc.subcore_barrier()
    pltpu.sync_copy(fpk_sh.at[pl.ds(sid * FSZ, FLEN)], fpkv)

    def p4(i, _):
        v = fpkv[pl.ds(i * L, L)]
        fsrc_v[pl.ds(i * L, L)] = jnp.minimum(v & 16383, N - 1)
        fdst_v[pl.ds(i * L, L)] = jnp.minimum(
            lax.shift_right_logical(v, 14), ACC_ROWS - 1)
        return ()

    lax.fori_loop(0, FLEN // L, p4, ())

    for k in range(512 // L):
        fsrc_v[pl.ds(off + k * L, L)] = (iota * 613 + k * 131) % N
        fdst_v[pl.ds(off + k * L, L)] = BS + ((iota + k * L) % CH)

    pltpu.sync_copy(fsrc_v, fpk_sh.at[pl.ds(sid * FSZ, FLEN)])
    pltpu.sync_copy(fpk_sh.at[pl.ds(sid * FSZ, FLEN)], fsrc_v)
    pltpu.sync_copy(fdst_v, fpk_sh.at[pl.ds(sid * FSZ, FLEN)])
    pltpu.sync_copy(fpk_sh.at[pl.ds(sid * FSZ, FLEN)], fdst_v)

    npairs = lax.shift_right_logical(off + 127, 7)

    plsc.subcore_barrier()

    pltpu.async_copy(xs_hbm.at[fsrc_v.at[pl.ds(0, CHS)]], buf0, sem0)
    pltpu.async_copy(xs_hbm.at[fsrc_v.at[pl.ds(CHS, CHS)]], buf1, sem1)

    def body(i, _):
        q = i * (2 * CHS)
        pltpu.make_async_copy(xs_hbm.at[fsrc_v.at[pl.ds(q, CHS)]], buf0, sem0).wait()
        pltpu.sync_copy(buf0, acc_sh.at[fdst_v.at[pl.ds(q, CHS)]], add=True)
        pltpu.async_copy(xs_hbm.at[fsrc_v.at[pl.ds(q + 2 * CHS, CHS)]], buf0, sem0)
        pltpu.make_async_copy(xs_hbm.at[fsrc_v.at[pl.ds(q + CHS, CHS)]], buf1, sem1).wait()
        pltpu.sync_copy(buf1, acc_sh.at[fdst_v.at[pl.ds(q + CHS, CHS)]], add=True)
        pltpu.async_copy(xs_hbm.at[fsrc_v.at[pl.ds(q + 3 * CHS, CHS)]], buf1, sem1)
        return ()

    lax.fori_loop(0, npairs, body, ())
    pltpu.make_async_copy(xs_hbm.at[fsrc_v.at[pl.ds(0, CHS)]], buf0, sem0).wait()
    pltpu.make_async_copy(xs_hbm.at[fsrc_v.at[pl.ds(0, CHS)]], buf1, sem1).wait()
    plsc.subcore_barrier()
    orows = BS // NS
    for k in range(orows // CHS):
        pltpu.sync_copy(acc_sh.at[pl.ds(sid * orows + k * CHS, CHS)], buf0)
        pltpu.sync_copy(buf0, out_hbm.at[cid, pl.ds(sid * orows + k * CHS, CHS)])


@functools.cache
def _sc_kernels():
    mesh = plsc.VectorSubcoreMesh(core_axis_name="c", subcore_axis_name="s",
                                  num_cores=NC, num_subcores=NS)
    sc_hist = pl.kernel(
        _sc_hist_body,
        out_type=jax.ShapeDtypeStruct((NC, HIST_ROWS), jnp.float32),
        mesh=mesh,
        scratch_types=[
            pltpu.VMEM((NCHUNK, CH), jnp.int32),
            pltpu.VMEM((CH,), jnp.float32),
            pltpu.VMEM((HIST_ROWS // NS,), jnp.float32),
            pltpu.VMEM_SHARED((HIST_ROWS,), jnp.float32),
        ],
    )
    sc_scatter = pl.kernel(
        _sc_scatter_body,
        out_type=jax.ShapeDtypeStruct((NC, BS, HP), jnp.float32),
        mesh=mesh,
        scratch_types=[
            pltpu.VMEM((NCHUNK, CH), jnp.int32),
            pltpu.VMEM((NCHUNK, CH), jnp.int32),
            pltpu.VMEM((EPW,), jnp.int32),
            pltpu.VMEM((FLEN,), jnp.int32),
            pltpu.VMEM((FLEN,), jnp.int32),
            pltpu.VMEM((FLEN,), jnp.int32),
            pltpu.VMEM((ZB,), jnp.int32),
            pltpu.VMEM((CHS, HP), jnp.float32),
            pltpu.VMEM((CHS, HP), jnp.float32),
            pltpu.VMEM((2 * 16,), jnp.int32),
            pltpu.SMEM((16,), jnp.int32),
            pltpu.VMEM_SHARED((NS * FSZ,), jnp.int32),
            pltpu.VMEM_SHARED((ACC_ROWS, HP), jnp.float32),
            pltpu.SemaphoreType.DMA,
            pltpu.SemaphoreType.DMA,
        ],
    )
    return sc_hist, sc_scatter


def _tc1a_body(x_ref, w1_ref, b1_ref, w2_ref, b2_ref, wg_ref, xw_ref):
    x = x_ref[...]
    z = jnp.dot(x, w1_ref[...], preferred_element_type=jnp.float32) + b1_ref[...]
    h1 = jnp.where(z >= 0, z, 0.01 * z)
    h = jnp.dot(h1, w2_ref[...], preferred_element_type=jnp.float32) + b2_ref[...]
    xw_ref[...] = jnp.dot(h, wg_ref[...], preferred_element_type=jnp.float32)


def _tc1a(x, W1, b1, W2, b2, Wg):
    R = 1000
    return pl.pallas_call(
        _tc1a_body,
        grid=(N // R,),
        in_specs=[
            pl.BlockSpec((R, IN_CH), lambda i: (i, 0)),
            pl.BlockSpec((IN_CH, HID), lambda i: (0, 0)),
            pl.BlockSpec((1, HID), lambda i: (0, 0)),
            pl.BlockSpec((HID, HID), lambda i: (0, 0)),
            pl.BlockSpec((1, HID), lambda i: (0, 0)),
            pl.BlockSpec((HID, HID), lambda i: (0, 0)),
        ],
        out_specs=[pl.BlockSpec((R, HID), lambda i: (i, 0))],
        out_shape=[jax.ShapeDtypeStruct((N, HID), jnp.float32)],
    )(x, W1, b1.reshape(1, HID), W2, b2.reshape(1, HID), Wg)[0]


def _tc1b_body(xw_ref, dp0_ref, dp1_ref, xs_ref, dinv_ref):
    xw = xw_ref[...]
    deg = dp0_ref[...] + dp1_ref[...] + 1.0
    dinv = lax.rsqrt(deg)
    xsp = jnp.concatenate(
        [xw * dinv, jnp.zeros((xw.shape[0], HP - HID), jnp.float32)], axis=1)
    xs_ref[...] = xsp
    dinv_ref[...] = dinv


def _tc1b(xw, dp0, dp1):
    R = 1000
    return pl.pallas_call(
        _tc1b_body,
        grid=(N // R,),
        in_specs=[
            pl.BlockSpec((R, HID), lambda i: (i, 0)),
            pl.BlockSpec((R, 1), lambda i: (i, 0)),
            pl.BlockSpec((R, 1), lambda i: (i, 0)),
        ],
        out_specs=[
            pl.BlockSpec((R, HP), lambda i: (i, 0)),
            pl.BlockSpec((R, 1), lambda i: (i, 0)),
        ],
        out_shape=[
            jax.ShapeDtypeStruct((N, HP), jnp.float32),
            jax.ShapeDtypeStruct((N, 1), jnp.float32),
        ],
    )(xw, dp0, dp1)


def _tc2_body(a0_ref, a1_ref, xs_ref, dinv_ref, bg_ref,
              wrb_ref, brb_ref, wrr_ref, brr_ref, wbb_ref, bbb_ref,
              qw_ref, cw_ref, cb_ref, yhat_ref, hb_ref):
    hb = (dinv_ref[...]
          * (a0_ref[...][:, :HID] + a1_ref[...][:, :HID] + xs_ref[...][:, :HID])
          + bg_ref[...])
    br = hb[:, :DIM]
    rr = hb[:, DIM:2 * DIM]
    bb = hb[:, 2 * DIM:]
    w0 = jnp.dot(jnp.tanh(br) @ wrb_ref[...] + brb_ref[...], qw_ref[...],
                 preferred_element_type=jnp.float32)
    w1 = jnp.dot(jnp.tanh(rr) @ wrr_ref[...] + brr_ref[...], qw_ref[...],
                 preferred_element_type=jnp.float32)
    w2 = jnp.dot(jnp.tanh(bb) @ wbb_ref[...] + bbb_ref[...], qw_ref[...],
                 preferred_element_type=jnp.float32)
    m = jnp.maximum(jnp.maximum(w0, w1), w2)
    e0 = jnp.exp(w0 - m)
    e1 = jnp.exp(w1 - m)
    e2 = jnp.exp(w2 - m)
    s = e0 + e1 + e2
    cx = (e0 * br + e1 * rr + e2 * bb) / s
    yhat_ref[...] = jnp.dot(cx, cw_ref[...], preferred_element_type=jnp.float32) + cb_ref[...]
    hb_ref[...] = hb


def _tc2(a0, a1, xsb, dinvb, bg, Wrb_w, Wrb_b, Wrr_w, Wrr_b, Wbb_w, Wbb_b,
         q_w, cls_w, cls_b):
    R = 1024
    grid = (BS // R,)
    full = lambda i: (0, 0)
    row = lambda i: (i, 0)
    return pl.pallas_call(
        _tc2_body,
        grid=grid,
        in_specs=[
            pl.BlockSpec((R, HP), row),
            pl.BlockSpec((R, HP), row),
            pl.BlockSpec((R, HP), row),
            pl.BlockSpec((R, 1), row),
            pl.BlockSpec((1, HID), full),
            pl.BlockSpec((DIM, DIM), full),
            pl.BlockSpec((1, DIM), full),
            pl.BlockSpec((DIM, DIM), full),
            pl.BlockSpec((1, DIM), full),
            pl.BlockSpec((DIM, DIM), full),
            pl.BlockSpec((1, DIM), full),
            pl.BlockSpec((DIM, 1), full),
            pl.BlockSpec((DIM, 2), full),
            pl.BlockSpec((1, 2), full),
        ],
        out_specs=[
            pl.BlockSpec((R, 2), row),
            pl.BlockSpec((R, HID), row),
        ],
        out_shape=[
            jax.ShapeDtypeStruct((BS, 2), jnp.float32),
            jax.ShapeDtypeStruct((BS, HID), jnp.float32),
        ],
    )(a0, a1, xsb, dinvb, bg.reshape(1, HID),
      Wrb_w, Wrb_b.reshape(1, DIM), Wrr_w, Wrr_b.reshape(1, DIM),
      Wbb_w, Wbb_b.reshape(1, DIM), q_w, cls_w, cls_b.reshape(1, 2))


def kernel(train, batch_size, x, edge_index, y, W1, b1, W2, b2, Wg, bg,
           Wrb_w, Wrb_b, Wrr_w, Wrr_b, Wbb_w, Wbb_b, q_w, cls_w, cls_b):
    pad_n = NW * EPW - E
    pad_src = (jnp.arange(pad_n, dtype=jnp.int32) * 131) % N
    src_p = jnp.concatenate([edge_index[0], pad_src]).reshape(NW, NCHUNK, CH)
    dst_all = jnp.concatenate(
        [edge_index[1], jnp.full((pad_n,), TRASH, jnp.int32)])
    dst_p = dst_all.reshape(NW, NCHUNK, CH)
    spread = BS + (jnp.arange(NW * EPW, dtype=jnp.int32) % 128)
    dst_s = jnp.where(dst_all < BS, dst_all, spread).reshape(NW, NCHUNK, CH)
    ones1 = jnp.ones((CH,), jnp.float32)
    zeros1 = jnp.zeros((HIST_ROWS // NS,), jnp.float32)
    zeros96 = jnp.zeros((CHS, HP), jnp.float32)
    zi32 = jnp.zeros((ZB,), jnp.int32)

    sc_hist, sc_scatter = _sc_kernels()
    xw = _tc1a(x, W1, b1, W2, b2, Wg)
    hist = sc_hist(dst_p, ones1, zeros1)
    dp0 = hist[0, :N, None]
    dp1 = hist[1, :N, None]
    xs, dinv = _tc1b(xw, dp0, dp1)
    acc = sc_scatter(xs, src_p, dst_s, zeros96, zi32)
    yhat, hb = _tc2(acc[0], acc[1], xs[:BS], dinv[:BS], bg,
                    Wrb_w, Wrb_b, Wrr_w, Wrr_b, Wbb_w, Wbb_b,
                    q_w, cls_w, cls_b)
    return (yhat, hb[:, :DIM], hb[:, DIM:])
